# baseline (speedup 1.0000x reference)
import functools

import jax
import jax.numpy as jnp
from jax import lax
from jax.experimental import pallas as pl
from jax.experimental.pallas import tpu as pltpu
from jax.experimental.pallas import tpu_sc as plsc

EMBED = 64
SCALE = float(EMBED) ** 0.5

NC = 2
NS = 16
NW = NC * NS
BB = 128


def _make_kernel(b, s):
    assert b == NW * BB and EMBED == 64

    mesh = plsc.VectorSubcoreMesh(core_axis_name="c", subcore_axis_name="s")
    row_bytes = BB * EMBED * 4

    @functools.partial(
        pl.kernel,
        mesh=mesh,
        out_type=jax.ShapeDtypeStruct((s, 8, NW, 8, BB), jnp.float32),
        scratch_types=[
            pltpu.VMEM((s, BB), jnp.int32),
            pltpu.VMEM((2, BB, EMBED), jnp.float32),
            pltpu.VMEM((2, 8, 8, BB), jnp.float32),
            pltpu.SemaphoreType.DMA,
            pltpu.SemaphoreType.DMA,
        ],
        compiler_params=pltpu.CompilerParams(
            use_tc_tiling_on_sc=False, needs_layout_passes=False),
    )
    def k(idx_hbm, table_hbm, out_hbm, idx_v, rows, tiles, gsem, ssem):
        wid = lax.axis_index("s") * NC + lax.axis_index("c")

        pltpu.sync_copy(idx_hbm.at[:, pl.ds(wid * BB, BB)], idx_v)

        def start_gather(step, p):
            pltpu.async_copy(table_hbm.at[idx_v.at[step]], rows.at[p], gsem)

        def wait_gather(p):
            pltpu.make_async_copy(
                table_hbm.at[pl.ds(0, BB)], rows.at[p], gsem).wait()

        iota = lax.iota(jnp.int32, 16)

        def transpose_scale(p):
            def kbody(kk, c):
                bvec = iota + kk * 16
                for e in range(EMBED):
                    evec = jnp.full((16,), e, jnp.int32)
                    v = plsc.load_gather(rows.at[p], [bvec, evec])
                    tiles[p, e // 8, e % 8, pl.ds(kk * 16, 16)] = v * SCALE
                return c

            lax.fori_loop(0, BB // 16, kbody, 0)

        def start_store(step, p):
            for a in range(8):
                pltpu.async_copy(
                    tiles.at[p, a], out_hbm.at[step, a, wid], ssem)

        def wait_store(p):
            for a in range(8):
                pltpu.make_async_copy(
                    tiles.at[p, a], out_hbm.at[0, a, wid], ssem).wait()

        start_gather(0, 0)
        start_gather(1, 1)

        def outer(i, carry):
            s0 = i * 2
            for p in range(2):
                step = s0 + p
                wait_gather(p)

                @pl.when(s0 >= 2)
                def _():
                    wait_store(p)

                transpose_scale(p)
                start_store(step, p)
                start_gather(jnp.minimum(step + 2, s - 1), p)
            return carry

        lax.fori_loop(0, s // 2, outer, 0)
        wait_store(0)
        wait_store(1)
        wait_gather(0)
        wait_gather(1)

    return k


def kernel(word_ids, table):
    b, s = word_ids.shape
    out5 = _make_kernel(b, s)(word_ids.T, table)
    return out5.transpose(2, 4, 0, 1, 3).reshape(b, s, EMBED)


# speedup vs baseline: 1.5508x; 1.5508x over previous
import functools

import jax
import jax.numpy as jnp
from jax import lax
from jax.experimental import pallas as pl
from jax.experimental.pallas import tpu as pltpu
from jax.experimental.pallas import tpu_sc as plsc

EMBED = 64
SCALE = float(EMBED) ** 0.5

NC = 2
NS = 16
NW = NC * NS
BB = 128


def _make_kernel(b, s):
    assert b == NW * BB and EMBED == 64

    mesh = plsc.VectorSubcoreMesh(core_axis_name="c", subcore_axis_name="s")
    row_bytes = BB * EMBED * 4

    @functools.partial(
        pl.kernel,
        mesh=mesh,
        out_type=jax.ShapeDtypeStruct((s, 8, NW, 8, BB), jnp.float32),
        scratch_types=[
            pltpu.VMEM((s, BB), jnp.int32),
            pltpu.VMEM((2, BB, EMBED), jnp.float32),
            pltpu.VMEM((2, 8, 8, BB), jnp.float32),
            pltpu.SemaphoreType.DMA,
            pltpu.SemaphoreType.DMA,
        ],
        compiler_params=pltpu.CompilerParams(
            use_tc_tiling_on_sc=False, needs_layout_passes=False),
    )
    def k(idx_hbm, table_hbm, out_hbm, idx_v, rows, tiles, gsem, ssem):
        wid = lax.axis_index("s") * NC + lax.axis_index("c")

        pltpu.sync_copy(idx_hbm.at[:, pl.ds(wid * BB, BB)], idx_v)

        def start_gather(step, p):
            pltpu.async_copy(table_hbm.at[idx_v.at[step]], rows.at[p], gsem)

        def wait_gather(p):
            pltpu.make_async_copy(
                table_hbm.at[pl.ds(0, BB)], rows.at[p], gsem).wait()

        iota = lax.iota(jnp.int32, 16)

        def transpose_scale(p):
            rp = rows.at[p]

            @plsc.parallel_loop(0, (BB // 16) * EMBED, 1, unroll=8)
            def _(i):
                kk = i >> 6
                e = i & 63
                bvec = iota + (kk << 4)
                evec = jnp.full((16,), 0, jnp.int32) + e
                v = plsc.load_gather(rp, [bvec, evec])
                tiles[p, e >> 3, e & 7, pl.ds(kk * 16, 16)] = v * SCALE

        def start_store(step, p):
            for a in range(8):
                pltpu.async_copy(
                    tiles.at[p, a], out_hbm.at[step, a, wid], ssem)

        def wait_store(p):
            for a in range(8):
                pltpu.make_async_copy(
                    tiles.at[p, a], out_hbm.at[0, a, wid], ssem).wait()

        start_gather(0, 0)
        start_gather(1, 1)

        def outer(i, carry):
            s0 = i * 2
            for p in range(2):
                step = s0 + p
                wait_gather(p)

                @pl.when(s0 >= 2)
                def _():
                    wait_store(p)

                transpose_scale(p)
                start_store(step, p)
                start_gather(jnp.minimum(step + 2, s - 1), p)
            return carry

        lax.fori_loop(0, s // 2, outer, 0)
        wait_store(0)
        wait_store(1)
        wait_gather(0)
        wait_gather(1)

    return k


def kernel(word_ids, table):
    b, s = word_ids.shape
    out5 = _make_kernel(b, s)(word_ids.T, table)
    return out5.transpose(2, 4, 0, 1, 3).reshape(b, s, EMBED)
